# baseline (speedup 1.0000x reference)
import functools
import math

import jax
import jax.numpy as jnp
from jax import lax
from jax.experimental import pallas as pl
from jax.experimental.pallas import tpu as pltpu
from jax.experimental.pallas import tpu_sc as plsc

D_EMBED = 32
VOCAB = 1000000
BATCH = 16384
HIST = 20
SCALE = math.sqrt(D_EMBED)

NC, NS, L = 2, 16, 16
NW = NC * NS
BPW = BATCH // NW
IDXR = BPW // 128


def _emb_body(x_hbm, table_hbm, out_hbm, idx_v, rows_v, tr_v, sem):
    wid = lax.axis_index("s") * NC + lax.axis_index("c")
    b0 = wid * BPW

    lane = lax.iota(jnp.int32, L)

    @pl.loop(0, HIST)
    def _h(h):
        pltpu.sync_copy(x_hbm.at[h, pl.ds(wid * IDXR, IDXR)], idx_v)

        cps = []
        for r in range(IDXR):
            cps.append(pltpu.async_copy(
                table_hbm.at[idx_v.at[r]],
                rows_v.at[pl.ds(r * 128, 128)],
                sem,
            ))
        for cp in cps:
            cp.wait()

        @pl.loop(0, D_EMBED)
        def _d(d):
            col = jnp.full((L,), 0, jnp.int32) + d

            @pl.loop(0, BPW // L, unroll=4)
            def _j(j):
                v = plsc.load_gather(rows_v, [j * L + lane, col])
                tr_v[d, pl.ds(j * L, L)] = v * SCALE

        pltpu.sync_copy(tr_v, out_hbm.at[h, :, pl.ds(b0, BPW)])


def _emb(x3d, table):
    mesh = plsc.VectorSubcoreMesh(core_axis_name="c", subcore_axis_name="s")
    f = functools.partial(
        pl.kernel,
        out_type=jax.ShapeDtypeStruct((HIST, D_EMBED, BATCH), jnp.float32),
        mesh=mesh,
        scratch_types=[
            pltpu.VMEM((IDXR, 128), jnp.int32),
            pltpu.VMEM((BPW, D_EMBED), jnp.float32),
            pltpu.VMEM((D_EMBED, BPW), jnp.float32),
            pltpu.SemaphoreType.DMA,
        ],
        compiler_params=pltpu.CompilerParams(
            use_tc_tiling_on_sc=False, needs_layout_passes=False),
    )(_emb_body)
    return f(x3d, table)


def kernel(x, table):
    x3d = x.T.reshape(HIST, BATCH // 128, 128)
    out_t = _emb(x3d, table)
    return out_t.transpose(2, 0, 1)
